# Initial kernel scaffold; baseline (speedup 1.0000x reference)
import functools

import jax
import jax.numpy as jnp
from jax import lax
from jax.experimental import pallas as pl
from jax.experimental.pallas import tpu as pltpu
from jax.experimental.pallas import tpu_sc as plsc

NUM_NODES = 100000
NUM_ITEMS = 100000
NUM_RATINGS = 5
HIST = 50
D = 64
B = 16384

NC = 2
NS = 16
L = 16
NW = NC * NS
PER_W = B // NW
NB = 16
N_CHUNKS = PER_W // NB

_J = D // L


def _tables_kernel(v2e, wg_top, r2e8, wg_bot, bg8):
    RB = 2000

    def body(v_ref, wt_ref, r8_ref, wb_ref, bg_ref, tv_ref, tr_ref):
        tv_ref[...] = jnp.dot(v_ref[...], wt_ref[...],
                              preferred_element_type=jnp.float32)

        @pl.when(pl.program_id(0) == 0)
        def _():
            tr_ref[...] = jnp.dot(r8_ref[...], wb_ref[...],
                                  preferred_element_type=jnp.float32) + bg_ref[...]

    return pl.pallas_call(
        body,
        grid=(NUM_ITEMS // RB,),
        in_specs=[
            pl.BlockSpec((RB, D), lambda i: (i, 0)),
            pl.BlockSpec((D, D), lambda i: (0, 0)),
            pl.BlockSpec((8, D), lambda i: (0, 0)),
            pl.BlockSpec((D, D), lambda i: (0, 0)),
            pl.BlockSpec((8, D), lambda i: (0, 0)),
        ],
        out_specs=[
            pl.BlockSpec((RB, D), lambda i: (i, 0)),
            pl.BlockSpec((8, D), lambda i: (0, 0)),
        ],
        out_shape=[
            jax.ShapeDtypeStruct((NUM_ITEMS, D), jnp.float32),
            jax.ShapeDtypeStruct((8, D), jnp.float32),
        ],
    )(v2e, wg_top, r2e8, wg_bot, bg8)


def _sc_aggregate(nodes, history_uv, history_uv_mask, history_r, tv, tr8,
                  feat_table):
    mesh = plsc.VectorSubcoreMesh(core_axis_name="c", subcore_axis_name="s",
                                  num_cores=NC, num_subcores=NS)

    @functools.partial(
        pl.kernel,
        out_type=(jax.ShapeDtypeStruct((B, D), jnp.float32),
                  jax.ShapeDtypeStruct((B, D), jnp.float32)),
        mesh=mesh,
        scratch_types=[
            pltpu.VMEM((NB,), jnp.int32),
            pltpu.VMEM((NB, HIST), jnp.int32),
            pltpu.VMEM((NB, HIST), jnp.int32),
            pltpu.VMEM((NB, HIST), jnp.int32),
            pltpu.VMEM((NB * HIST, D), jnp.float32),
            pltpu.VMEM((8, D), jnp.float32),
            pltpu.VMEM((NB, D), jnp.float32),
            pltpu.VMEM((NB, D), jnp.float32),
            pltpu.SemaphoreType.DMA,
            pltpu.SemaphoreType.DMA,
        ],
    )
    def k(nodes_h, huv_h, m_h, hr_h, tv_h, tr_h, feat_h,
          neigh_o, feat_o,
          nid_v, huv_v, m_v, hr_v, rows_v, tr_v, feat_v, out_v, sem, sem2):
        wid = lax.axis_index("s") * NC + lax.axis_index("c")
        pltpu.sync_copy(tr_h, tr_v)
        iota = lax.broadcasted_iota(jnp.int32, (L,), 0)
        tail_f = (iota >= (3 * L - (HIST - L))).astype(jnp.float32)

        def chunk(c, carry):
            base = wid * PER_W + c * NB
            pltpu.sync_copy(nodes_h.at[pl.ds(base, NB)], nid_v)
            cp_uv = pltpu.async_copy(huv_h.at[nid_v], huv_v, sem)
            cp_m = pltpu.async_copy(m_h.at[nid_v], m_v, sem)
            cp_r = pltpu.async_copy(hr_h.at[nid_v], hr_v, sem)
            cp_f = pltpu.async_copy(feat_h.at[nid_v], feat_v, sem2)
            cp_uv.wait()
            cp_m.wait()
            cp_r.wait()
            tv_cps = [
                pltpu.async_copy(tv_h.at[huv_v.at[n]],
                                 rows_v.at[pl.ds(n * HIST, HIST)], sem)
                for n in range(NB)
            ]
            for cp in tv_cps:
                cp.wait()

            def node(n, carry2):
                fn = jnp.full((L,), n, jnp.int32)
                m0 = plsc.load_gather(m_v, [fn, iota]).astype(jnp.float32)
                m1 = plsc.load_gather(m_v, [fn, iota + L]).astype(jnp.float32)
                m2 = plsc.load_gather(m_v, [fn, iota + 2 * L]).astype(jnp.float32)
                m3 = plsc.load_gather(m_v, [fn, iota + (HIST - L)]).astype(jnp.float32)
                denom = jnp.maximum(jnp.sum(m0 + m1 + m2 + m3 * tail_f), 1.0)

                row0 = n * HIST
                acc = [jnp.zeros((L,), jnp.float32) for _ in range(_J)]
                for h in range(HIST):
                    fh = jnp.full((L,), h, jnp.int32)
                    rb = plsc.load_gather(hr_v, [fn, fh])
                    mb = plsc.load_gather(m_v, [fn, fh]).astype(jnp.float32)
                    for j in range(_J):
                        tvj = rows_v[row0 + h, pl.ds(j * L, L)]
                        trj = plsc.load_gather(tr_v, [rb, iota + j * L])
                        acc[j] = acc[j] + jnp.maximum(tvj + trj, 0.0) * mb
                inv = 1.0 / denom
                for j in range(_J):
                    out_v[n, pl.ds(j * L, L)] = acc[j] * inv
                return carry2

            lax.fori_loop(0, NB, node, 0)
            cp_f.wait()
            pltpu.sync_copy(out_v, neigh_o.at[pl.ds(base, NB)])
            pltpu.sync_copy(feat_v, feat_o.at[pl.ds(base, NB)])
            return carry

        lax.fori_loop(0, N_CHUNKS, chunk, 0)

    return k(nodes, history_uv, history_uv_mask, history_r, tv, tr8,
             feat_table)


def _final_linear(feats, neigh, w1a, w1b, b1r):
    RB = 2048

    def body(f_ref, n_ref, wa_ref, wb_ref, b_ref, o_ref):
        y = jnp.dot(f_ref[...], wa_ref[...], preferred_element_type=jnp.float32)
        y = y + jnp.dot(n_ref[...], wb_ref[...],
                        preferred_element_type=jnp.float32)
        o_ref[...] = jnp.maximum(y + b_ref[...], 0.0)

    return pl.pallas_call(
        body,
        grid=(B // RB,),
        in_specs=[
            pl.BlockSpec((RB, D), lambda i: (i, 0)),
            pl.BlockSpec((RB, D), lambda i: (i, 0)),
            pl.BlockSpec((D, D), lambda i: (0, 0)),
            pl.BlockSpec((D, D), lambda i: (0, 0)),
            pl.BlockSpec((1, D), lambda i: (0, 0)),
        ],
        out_specs=pl.BlockSpec((RB, D), lambda i: (i, 0)),
        out_shape=jax.ShapeDtypeStruct((B, D), jnp.float32),
    )(feats, neigh, w1a, w1b, b1r)


def kernel(nodes, history_uv, history_uv_mask, history_r, feat_table, v2e,
           r2e, Wg, bg, W1, b1):
    wg_top, wg_bot = Wg[:D], Wg[D:]
    r2e8 = jnp.zeros((8, D), r2e.dtype).at[:NUM_RATINGS].set(r2e)
    bg8 = jnp.broadcast_to(bg[None, :], (8, D))
    tv, tr8 = _tables_kernel(v2e, wg_top, r2e8, wg_bot, bg8)
    neigh, feats = _sc_aggregate(nodes, history_uv, history_uv_mask,
                                 history_r, tv, tr8, feat_table)
    return _final_linear(feats, neigh, W1[:D], W1[D:], b1.reshape(1, D))


# speedup vs baseline: 4.5400x; 4.5400x over previous
import functools

import jax
import jax.numpy as jnp
from jax import lax
from jax.experimental import pallas as pl
from jax.experimental.pallas import tpu as pltpu
from jax.experimental.pallas import tpu_sc as plsc

NUM_NODES = 100000
NUM_ITEMS = 100000
NUM_RATINGS = 5
HIST = 50
D = 64
B = 16384

NC = 2
NS = 16
L = 16
NW = NC * NS
PER_W = B // NW
NB = 16
N_CHUNKS = PER_W // NB

_J = D // L


def _tables_kernel(v2e, wg_top, r2e8, wg_bot, bg8):
    RB = min(2000, NUM_ITEMS)

    def body(v_ref, wt_ref, r8_ref, wb_ref, bg_ref, tv_ref, tr_ref):
        tv_ref[...] = jnp.dot(v_ref[...], wt_ref[...],
                              preferred_element_type=jnp.float32,
                              precision=lax.Precision.HIGHEST)

        @pl.when(pl.program_id(0) == 0)
        def _():
            tr_ref[...] = jnp.dot(r8_ref[...], wb_ref[...],
                                  preferred_element_type=jnp.float32,
                              precision=lax.Precision.HIGHEST) + bg_ref[...]

    return pl.pallas_call(
        body,
        grid=(NUM_ITEMS // RB,),
        in_specs=[
            pl.BlockSpec((RB, D), lambda i: (i, 0)),
            pl.BlockSpec((D, D), lambda i: (0, 0)),
            pl.BlockSpec((8, D), lambda i: (0, 0)),
            pl.BlockSpec((D, D), lambda i: (0, 0)),
            pl.BlockSpec((8, D), lambda i: (0, 0)),
        ],
        out_specs=[
            pl.BlockSpec((RB, D), lambda i: (i, 0)),
            pl.BlockSpec((8, D), lambda i: (0, 0)),
        ],
        out_shape=[
            jax.ShapeDtypeStruct((NUM_ITEMS, D), jnp.float32),
            jax.ShapeDtypeStruct((8, D), jnp.float32),
        ],
    )(v2e, wg_top, r2e8, wg_bot, bg8)


def _sc_aggregate(nodes, huv2, m2, hr2, tv, tr8, feat_table):
    mesh = plsc.VectorSubcoreMesh(core_axis_name="c", subcore_axis_name="s",
                                  num_cores=NC, num_subcores=NS)

    @functools.partial(
        pl.kernel,
        out_type=(jax.ShapeDtypeStruct((B, D), jnp.float32),
                  jax.ShapeDtypeStruct((B, D), jnp.float32)),
        mesh=mesh,
        compiler_params=pltpu.CompilerParams(needs_layout_passes=False,
                                             use_tc_tiling_on_sc=False),
        scratch_types=[
            pltpu.VMEM((NB,), jnp.int32),
            pltpu.VMEM((L,), jnp.int32),
            pltpu.VMEM((L,), jnp.int32),
            pltpu.VMEM((_J * NB,), jnp.int32),
            pltpu.VMEM((_J * NB, L), jnp.int32),
            pltpu.VMEM((_J * NB, L), jnp.int32),
            pltpu.VMEM((_J * NB, L), jnp.int32),
            pltpu.VMEM((8, D), jnp.float32),
            pltpu.VMEM((NB, D), jnp.float32),
            pltpu.VMEM((NB, D), jnp.float32),
            pltpu.VMEM((L, D), jnp.float32),
            pltpu.VMEM((L,), jnp.int32),
            pltpu.SemaphoreType.DMA,
            pltpu.SemaphoreType.DMA,
        ],
    )
    def k(nodes_h, huv_h, m_h, hr_h, tv_h, tr_h, feat_h,
          neigh_o, feat_o,
          nid_v, w0_v, d_v, hidx_v, huv_v, m_v, hr_v, tr_v, feat_v, out_v,
          wave_v, widx_v, sem, sem2):
        wid = lax.axis_index("s") * NC + lax.axis_index("c")
        pltpu.sync_copy(tr_h, tr_v)
        iota = lax.broadcasted_iota(jnp.int32, (L,), 0)
        tail_f = (iota >= 3 * L - (HIST - L)).astype(jnp.float32)

        def chunk(c, carry):
            base = wid * PER_W + c * NB
            pltpu.sync_copy(nodes_h.at[pl.ds(base, NB)], nid_v)
            nid = nid_v[...]
            cp_f = pltpu.async_copy(feat_h.at[nid_v], feat_v, sem2)
            p0 = nid * HIST
            w0_v[...] = jnp.right_shift(p0, 4)
            d_v[...] = jnp.bitwise_and(p0, L - 1)
            qsel = jnp.right_shift(iota, 2)
            wsel = jnp.bitwise_and(iota, _J - 1)
            for q in range(_J):
                hidx_v[pl.ds(q * L, L)] = (
                    plsc.load_gather(w0_v, [qsel + q * _J]) + wsel)
            cp_uv = pltpu.async_copy(huv_h.at[hidx_v], huv_v, sem)
            cp_m = pltpu.async_copy(m_h.at[hidx_v], m_v, sem)
            cp_r = pltpu.async_copy(hr_h.at[hidx_v], hr_v, sem)
            cp_uv.wait()
            cp_m.wait()
            cp_r.wait()

            def node(n, carry2):
                fn = jnp.full((L,), n, jnp.int32)
                dn = plsc.load_gather(d_v, [fn])
                pbase = fn * (_J * L) + dn

                dsum = jnp.zeros((L,), jnp.float32)
                for g, tf in ((0, None), (1, None), (2, None), (3, tail_f)):
                    off = g * L if g < _J - 1 else HIST - L
                    pw = pbase + iota + off
                    mw = plsc.load_gather(
                        m_v, [jnp.right_shift(pw, 4),
                              jnp.bitwise_and(pw, L - 1)]).astype(jnp.float32)
                    dsum = dsum + (mw if tf is None else mw * tf)
                denom = jnp.maximum(jnp.sum(dsum), 1.0)

                acc = [jnp.zeros((L,), jnp.float32) for _ in range(_J)]
                for g in range(_J):
                    off = g * L if g < _J - 1 else HIST - L
                    pw = pbase + iota + off
                    rw = jnp.right_shift(pw, 4)
                    cw = jnp.bitwise_and(pw, L - 1)
                    widx_v[...] = plsc.load_gather(huv_v, [rw, cw])
                    pltpu.async_copy(tv_h.at[widx_v], wave_v, sem).wait()
                    lo = 0 if g < _J - 1 else 3 * L - (HIST - L)
                    for hl in range(lo, L):
                        ph = pbase + (off + hl)
                        rh = jnp.right_shift(ph, 4)
                        ch = jnp.bitwise_and(ph, L - 1)
                        rb = plsc.load_gather(hr_v, [rh, ch])
                        mb = plsc.load_gather(m_v, [rh, ch]).astype(jnp.float32)
                        for j in range(_J):
                            tvj = wave_v[hl, pl.ds(j * L, L)]
                            trj = plsc.load_gather(tr_v, [rb, iota + j * L])
                            acc[j] = acc[j] + jnp.maximum(tvj + trj, 0.0) * mb
                inv = jnp.full((L,), 1.0, jnp.float32) / jnp.broadcast_to(denom, (L,))
                for j in range(_J):
                    out_v[n, pl.ds(j * L, L)] = acc[j] * inv
                return carry2

            lax.fori_loop(0, NB, node, 0)
            cp_f.wait()
            pltpu.sync_copy(out_v, neigh_o.at[pl.ds(base, NB)])
            pltpu.sync_copy(feat_v, feat_o.at[pl.ds(base, NB)])
            return carry

        lax.fori_loop(0, N_CHUNKS, chunk, 0)

    return k(nodes, huv2, m2, hr2, tv, tr8, feat_table)


def _final_linear(feats, neigh, w1a, w1b, b1r):
    RB = min(2048, B)

    def body(f_ref, n_ref, wa_ref, wb_ref, b_ref, o_ref):
        y = jnp.dot(f_ref[...], wa_ref[...], preferred_element_type=jnp.float32,
                              precision=lax.Precision.HIGHEST)
        y = y + jnp.dot(n_ref[...], wb_ref[...],
                        preferred_element_type=jnp.float32,
                              precision=lax.Precision.HIGHEST)
        o_ref[...] = jnp.maximum(y + b_ref[...], 0.0)

    return pl.pallas_call(
        body,
        grid=(B // RB,),
        in_specs=[
            pl.BlockSpec((RB, D), lambda i: (i, 0)),
            pl.BlockSpec((RB, D), lambda i: (i, 0)),
            pl.BlockSpec((D, D), lambda i: (0, 0)),
            pl.BlockSpec((D, D), lambda i: (0, 0)),
            pl.BlockSpec((1, D), lambda i: (0, 0)),
        ],
        out_specs=pl.BlockSpec((RB, D), lambda i: (i, 0)),
        out_shape=jax.ShapeDtypeStruct((B, D), jnp.float32),
    )(feats, neigh, w1a, w1b, b1r)


def kernel(nodes, history_uv, history_uv_mask, history_r, feat_table, v2e,
           r2e, Wg, bg, W1, b1):
    wg_top, wg_bot = Wg[:D], Wg[D:]
    r2e8 = jnp.zeros((8, D), r2e.dtype).at[:NUM_RATINGS].set(r2e)
    bg8 = jnp.broadcast_to(bg[None, :], (8, D))
    tv, tr8 = _tables_kernel(v2e, wg_top, r2e8, wg_bot, bg8)
    huv2 = history_uv.reshape(-1, L)
    m2 = history_uv_mask.reshape(-1, L)
    hr2 = history_r.reshape(-1, L)
    neigh, feats = _sc_aggregate(nodes, huv2, m2, hr2, tv, tr8, feat_table)
    return _final_linear(feats, neigh, W1[:D], W1[D:], b1.reshape(1, D))


# speedup vs baseline: 7.9297x; 1.7466x over previous
import functools

import jax
import jax.numpy as jnp
from jax import lax
from jax.experimental import pallas as pl
from jax.experimental.pallas import tpu as pltpu
from jax.experimental.pallas import tpu_sc as plsc

NUM_NODES = 100000
NUM_ITEMS = 100000
NUM_RATINGS = 5
HIST = 50
D = 64
B = 16384

NC = 2
NS = 16
L = 16
NW = NC * NS
PER_W = B // NW
NB = 16
N_CHUNKS = PER_W // NB

_J = D // L


def _tables_kernel(v2e, wg_top, r2e8, wg_bot, bg8):
    RB = min(2000, NUM_ITEMS)

    def body(v_ref, wt_ref, r8_ref, wb_ref, bg_ref, tv_ref, tr_ref):
        tv_ref[...] = jnp.dot(v_ref[...], wt_ref[...],
                              preferred_element_type=jnp.float32,
                              precision=lax.Precision.HIGHEST)

        @pl.when(pl.program_id(0) == 0)
        def _():
            tr_ref[...] = jnp.dot(r8_ref[...], wb_ref[...],
                                  preferred_element_type=jnp.float32,
                              precision=lax.Precision.HIGHEST) + bg_ref[...]

    return pl.pallas_call(
        body,
        grid=(NUM_ITEMS // RB,),
        in_specs=[
            pl.BlockSpec((RB, D), lambda i: (i, 0)),
            pl.BlockSpec((D, D), lambda i: (0, 0)),
            pl.BlockSpec((8, D), lambda i: (0, 0)),
            pl.BlockSpec((D, D), lambda i: (0, 0)),
            pl.BlockSpec((8, D), lambda i: (0, 0)),
        ],
        out_specs=[
            pl.BlockSpec((RB, D), lambda i: (i, 0)),
            pl.BlockSpec((8, D), lambda i: (0, 0)),
        ],
        out_shape=[
            jax.ShapeDtypeStruct((NUM_ITEMS, D), jnp.float32),
            jax.ShapeDtypeStruct((8, D), jnp.float32),
        ],
    )(v2e, wg_top, r2e8, wg_bot, bg8)


def _sc_aggregate(nodes, huv2, m2, hr2, tv, tr8, feat_table):
    mesh = plsc.VectorSubcoreMesh(core_axis_name="c", subcore_axis_name="s",
                                  num_cores=NC, num_subcores=NS)

    @functools.partial(
        pl.kernel,
        out_type=(jax.ShapeDtypeStruct((B, D), jnp.float32),
                  jax.ShapeDtypeStruct((B, D), jnp.float32)),
        mesh=mesh,
        compiler_params=pltpu.CompilerParams(needs_layout_passes=False,
                                             use_tc_tiling_on_sc=False),
        scratch_types=[
            pltpu.VMEM((NB,), jnp.int32),
            pltpu.VMEM((L,), jnp.int32),
            pltpu.VMEM((L,), jnp.int32),
            pltpu.VMEM((_J * NB,), jnp.int32),
            pltpu.VMEM((_J * NB, L), jnp.int32),
            pltpu.VMEM((_J * NB, L), jnp.int32),
            pltpu.VMEM((_J * NB, L), jnp.int32),
            pltpu.VMEM((8, D), jnp.float32),
            pltpu.VMEM((NB, D), jnp.float32),
            pltpu.VMEM((NB, D), jnp.float32),
            [pltpu.VMEM((L, D), jnp.float32) for _ in range(_J)],
            [pltpu.VMEM((L,), jnp.int32) for _ in range(_J)],
            pltpu.SemaphoreType.DMA,
            pltpu.SemaphoreType.DMA,
        ],
    )
    def k(nodes_h, huv_h, m_h, hr_h, tv_h, tr_h, feat_h,
          neigh_o, feat_o,
          nid_v, w0_v, d_v, hidx_v, huv_v, m_v, hr_v, tr_v, feat_v, out_v,
          wave_v, widx_v, sem, sem2):
        wid = lax.axis_index("s") * NC + lax.axis_index("c")
        pltpu.sync_copy(tr_h, tr_v)
        iota = lax.broadcasted_iota(jnp.int32, (L,), 0)
        tail_f = (iota >= 3 * L - (HIST - L)).astype(jnp.float32)

        def chunk(c, carry):
            base = wid * PER_W + c * NB
            pltpu.sync_copy(nodes_h.at[pl.ds(base, NB)], nid_v)
            nid = nid_v[...]
            cp_f = pltpu.async_copy(feat_h.at[nid_v], feat_v, sem2)
            p0 = nid * HIST
            w0_v[...] = jnp.right_shift(p0, 4)
            d_v[...] = jnp.bitwise_and(p0, L - 1)
            qsel = jnp.right_shift(iota, 2)
            wsel = jnp.bitwise_and(iota, _J - 1)
            for q in range(_J):
                hidx_v[pl.ds(q * L, L)] = (
                    plsc.load_gather(w0_v, [qsel + q * _J]) + wsel)
            cp_uv = pltpu.async_copy(huv_h.at[hidx_v], huv_v, sem)
            cp_m = pltpu.async_copy(m_h.at[hidx_v], m_v, sem)
            cp_r = pltpu.async_copy(hr_h.at[hidx_v], hr_v, sem)
            cp_uv.wait()
            cp_m.wait()
            cp_r.wait()

            def node(n, carry2):
                fn = jnp.full((L,), n, jnp.int32)
                dn = plsc.load_gather(d_v, [fn])
                pbase = fn * (_J * L) + dn

                dsum = jnp.zeros((L,), jnp.float32)
                for g, tf in ((0, None), (1, None), (2, None), (3, tail_f)):
                    off = g * L if g < _J - 1 else HIST - L
                    pw = pbase + iota + off
                    mw = plsc.load_gather(
                        m_v, [jnp.right_shift(pw, 4),
                              jnp.bitwise_and(pw, L - 1)]).astype(jnp.float32)
                    dsum = dsum + (mw if tf is None else mw * tf)
                denom = jnp.maximum(jnp.sum(dsum), 1.0)

                acc = [jnp.zeros((L,), jnp.float32) for _ in range(_J)]
                cps = []
                for g in range(_J):
                    off = g * L if g < _J - 1 else HIST - L
                    pw = pbase + iota + off
                    rw = jnp.right_shift(pw, 4)
                    cw = jnp.bitwise_and(pw, L - 1)
                    widx_v[g][...] = plsc.load_gather(huv_v, [rw, cw])
                    cps.append(
                        pltpu.async_copy(tv_h.at[widx_v[g]], wave_v[g], sem))
                for g in range(_J):
                    off = g * L if g < _J - 1 else HIST - L
                    cps[g].wait()
                    lo = 0 if g < _J - 1 else 3 * L - (HIST - L)
                    for hl in range(lo, L):
                        ph = pbase + (off + hl)
                        rh = jnp.right_shift(ph, 4)
                        ch = jnp.bitwise_and(ph, L - 1)
                        rb = plsc.load_gather(hr_v, [rh, ch])
                        mb = plsc.load_gather(m_v, [rh, ch]).astype(jnp.float32)
                        for j in range(_J):
                            tvj = wave_v[g][hl, pl.ds(j * L, L)]
                            trj = plsc.load_gather(tr_v, [rb, iota + j * L])
                            acc[j] = acc[j] + jnp.maximum(tvj + trj, 0.0) * mb
                inv = jnp.full((L,), 1.0, jnp.float32) / jnp.broadcast_to(denom, (L,))
                for j in range(_J):
                    out_v[n, pl.ds(j * L, L)] = acc[j] * inv
                return carry2

            lax.fori_loop(0, NB, node, 0)
            cp_f.wait()
            pltpu.sync_copy(out_v, neigh_o.at[pl.ds(base, NB)])
            pltpu.sync_copy(feat_v, feat_o.at[pl.ds(base, NB)])
            return carry

        lax.fori_loop(0, N_CHUNKS, chunk, 0)

    return k(nodes, huv2, m2, hr2, tv, tr8, feat_table)


def _final_linear(feats, neigh, w1a, w1b, b1r):
    RB = min(2048, B)

    def body(f_ref, n_ref, wa_ref, wb_ref, b_ref, o_ref):
        y = jnp.dot(f_ref[...], wa_ref[...], preferred_element_type=jnp.float32,
                              precision=lax.Precision.HIGHEST)
        y = y + jnp.dot(n_ref[...], wb_ref[...],
                        preferred_element_type=jnp.float32,
                              precision=lax.Precision.HIGHEST)
        o_ref[...] = jnp.maximum(y + b_ref[...], 0.0)

    return pl.pallas_call(
        body,
        grid=(B // RB,),
        in_specs=[
            pl.BlockSpec((RB, D), lambda i: (i, 0)),
            pl.BlockSpec((RB, D), lambda i: (i, 0)),
            pl.BlockSpec((D, D), lambda i: (0, 0)),
            pl.BlockSpec((D, D), lambda i: (0, 0)),
            pl.BlockSpec((1, D), lambda i: (0, 0)),
        ],
        out_specs=pl.BlockSpec((RB, D), lambda i: (i, 0)),
        out_shape=jax.ShapeDtypeStruct((B, D), jnp.float32),
    )(feats, neigh, w1a, w1b, b1r)


def kernel(nodes, history_uv, history_uv_mask, history_r, feat_table, v2e,
           r2e, Wg, bg, W1, b1):
    wg_top, wg_bot = Wg[:D], Wg[D:]
    r2e8 = jnp.zeros((8, D), r2e.dtype).at[:NUM_RATINGS].set(r2e)
    bg8 = jnp.broadcast_to(bg[None, :], (8, D))
    tv, tr8 = _tables_kernel(v2e, wg_top, r2e8, wg_bot, bg8)
    huv2 = history_uv.reshape(-1, L)
    m2 = history_uv_mask.reshape(-1, L)
    hr2 = history_r.reshape(-1, L)
    neigh, feats = _sc_aggregate(nodes, huv2, m2, hr2, tv, tr8, feat_table)
    return _final_linear(feats, neigh, W1[:D], W1[D:], b1.reshape(1, D))
